# Initial kernel scaffold; baseline (speedup 1.0000x reference)
import jax
import jax.numpy as jnp
from jax.experimental import pallas as pl


def kernel(x, edge_index, W1, b1, W2, b2, Wf1, bf1, Wf2, bf2):
    raise NotImplementedError("write your pallas kernel here")


# speedup vs baseline: 15.5325x; 15.5325x over previous
import functools

import jax
import jax.numpy as jnp
from jax import lax
from jax.experimental import pallas as pl
from jax.experimental.pallas import tpu as pltpu
from jax.experimental.pallas import tpu_sc as plsc

N_REAL = 10000
N_PAD = 10240
D = 128
E = 320000
NC = 2
NS = 16
NW = NC * NS
CHUNK = 128
CPT = 80
E_PAD = NW * CPT * CHUNK
RPT = N_PAD // NS
ZR = 64
BLK = 1024

_MESH = plsc.VectorSubcoreMesh(core_axis_name="c", subcore_axis_name="s")


@functools.partial(
    pl.kernel,
    mesh=_MESH,
    out_type=jax.ShapeDtypeStruct((NC * N_PAD, D), jnp.float32),
    scratch_types=[
        pltpu.VMEM((CHUNK,), jnp.int32),
        pltpu.VMEM((CHUNK, D), jnp.float32),
        pltpu.VMEM((ZR, D), jnp.float32),
        pltpu.VMEM_SHARED((N_PAD, D), jnp.float32),
    ],
)
def _deg_kernel(dst_hbm, ones_hbm, zeros_hbm, out_hbm,
                idx_v, ones_v, zbuf, acc_sh):
    c = lax.axis_index("c")
    s = lax.axis_index("s")
    wid = s * NC + c
    base = s * RPT

    pltpu.sync_copy(ones_hbm, ones_v)
    pltpu.sync_copy(zeros_hbm, zbuf)
    for k in range(RPT // ZR):
        pltpu.sync_copy(zbuf, acc_sh.at[pl.ds(base + k * ZR, ZR)])
    plsc.subcore_barrier()

    def body(j, _):
        pltpu.sync_copy(dst_hbm.at[pl.ds((wid * CPT + j) * CHUNK, CHUNK)],
                        idx_v)
        pltpu.sync_copy(ones_v, acc_sh.at[idx_v], add=True)
        return 0

    lax.fori_loop(0, CPT, body, 0)
    plsc.subcore_barrier()
    for k in range(RPT // ZR):
        pltpu.sync_copy(acc_sh.at[pl.ds(base + k * ZR, ZR)], zbuf)
        pltpu.sync_copy(zbuf, out_hbm.at[pl.ds(c * N_PAD + base + k * ZR, ZR)])


@functools.partial(
    pl.kernel,
    mesh=_MESH,
    out_type=jax.ShapeDtypeStruct((NC * N_PAD, D), jnp.float32),
    scratch_types=[
        pltpu.VMEM((CHUNK,), jnp.int32),
        pltpu.VMEM((CHUNK,), jnp.int32),
        pltpu.VMEM((CHUNK,), jnp.int32),
        pltpu.VMEM((CHUNK,), jnp.int32),
        pltpu.VMEM((CHUNK, D), jnp.float32),
        pltpu.VMEM((CHUNK, D), jnp.float32),
        pltpu.VMEM((ZR, D), jnp.float32),
        pltpu.VMEM_SHARED((N_PAD, D), jnp.float32),
        pltpu.SemaphoreType.DMA,
        pltpu.SemaphoreType.DMA,
    ],
)
def _prop_kernel(g_hbm, src_hbm, dst_hbm, zeros_hbm, out_hbm,
                 srcv0, srcv1, dstv0, dstv1, buf0, buf1, zbuf,
                 acc_sh, sem0, sem1):
    c = lax.axis_index("c")
    s = lax.axis_index("s")
    wid = s * NC + c
    base = s * RPT

    pltpu.sync_copy(zeros_hbm, zbuf)
    for k in range(RPT // ZR):
        pltpu.sync_copy(zbuf, acc_sh.at[pl.ds(base + k * ZR, ZR)])
    plsc.subcore_barrier()

    def body(j, _):
        e0 = (wid * CPT + 2 * j) * CHUNK
        pltpu.sync_copy(src_hbm.at[pl.ds(e0, CHUNK)], srcv0)
        pltpu.sync_copy(dst_hbm.at[pl.ds(e0, CHUNK)], dstv0)
        pltpu.sync_copy(src_hbm.at[pl.ds(e0 + CHUNK, CHUNK)], srcv1)
        pltpu.sync_copy(dst_hbm.at[pl.ds(e0 + CHUNK, CHUNK)], dstv1)
        h0 = pltpu.async_copy(g_hbm.at[srcv0], buf0, sem0)
        h1 = pltpu.async_copy(g_hbm.at[srcv1], buf1, sem1)
        h0.wait()
        pltpu.sync_copy(buf0, acc_sh.at[dstv0], add=True)
        h1.wait()
        pltpu.sync_copy(buf1, acc_sh.at[dstv1], add=True)
        return 0

    lax.fori_loop(0, CPT // 2, body, 0)
    plsc.subcore_barrier()
    for k in range(RPT // ZR):
        pltpu.sync_copy(acc_sh.at[pl.ds(base + k * ZR, ZR)], zbuf)
        pltpu.sync_copy(zbuf, out_hbm.at[pl.ds(c * N_PAD + base + k * ZR, ZR)])


def _stage_a(x_pad, deg, W1):
    def body(x_ref, deg_ref, w_ref, g_ref, dinv_ref):
        i = pl.program_id(0)
        deg_tot = deg_ref[0, :, 0:1] + deg_ref[1, :, 0:1] + 1.0
        rows = i * BLK + lax.broadcasted_iota(jnp.int32, (BLK, 1), 0)
        dinv = jnp.where(rows < N_REAL, lax.rsqrt(deg_tot), 0.0)
        g_ref[...] = jnp.dot(x_ref[...], w_ref[...],
                             preferred_element_type=jnp.float32) * dinv
        dinv_ref[...] = dinv

    return pl.pallas_call(
        body,
        grid=(N_PAD // BLK,),
        in_specs=[
            pl.BlockSpec((BLK, D), lambda i: (i, 0)),
            pl.BlockSpec((NC, BLK, D), lambda i: (0, i, 0)),
            pl.BlockSpec((D, D), lambda i: (0, 0)),
        ],
        out_specs=[
            pl.BlockSpec((BLK, D), lambda i: (i, 0)),
            pl.BlockSpec((BLK, 1), lambda i: (i, 0)),
        ],
        out_shape=[
            jax.ShapeDtypeStruct((N_PAD, D), jnp.float32),
            jax.ShapeDtypeStruct((N_PAD, 1), jnp.float32),
        ],
    )(x_pad, deg, W1)


def _stage_c(acc, g1, dinv, b1, W2):
    def body(acc_ref, g_ref, dinv_ref, b_ref, w_ref, out_ref):
        dv = dinv_ref[...]
        t = jnp.tanh((acc_ref[0] + acc_ref[1] + g_ref[...]) * dv + b_ref[...])
        out_ref[...] = jnp.dot(t, w_ref[...],
                               preferred_element_type=jnp.float32) * dv

    return pl.pallas_call(
        body,
        grid=(N_PAD // BLK,),
        in_specs=[
            pl.BlockSpec((NC, BLK, D), lambda i: (0, i, 0)),
            pl.BlockSpec((BLK, D), lambda i: (i, 0)),
            pl.BlockSpec((BLK, 1), lambda i: (i, 0)),
            pl.BlockSpec((1, D), lambda i: (0, 0)),
            pl.BlockSpec((D, D), lambda i: (0, 0)),
        ],
        out_specs=pl.BlockSpec((BLK, D), lambda i: (i, 0)),
        out_shape=jax.ShapeDtypeStruct((N_PAD, D), jnp.float32),
    )(acc, g1, dinv, b1, W2)


def _head(acc, g2, dinv, b2, Wf1, bf1, Wf2, bf2):
    def body(acc_ref, g_ref, dinv_ref, b2_ref, wf1_ref, bf1_ref,
             wf2_ref, bf2_ref, out_ref):
        dv = dinv_ref[...]
        t = jnp.tanh((acc_ref[0] + acc_ref[1] + g_ref[...]) * dv + b2_ref[...])
        f = jnp.tanh(jnp.dot(t, wf1_ref[...],
                             preferred_element_type=jnp.float32) + bf1_ref[...])
        out_ref[...] = jnp.dot(f, wf2_ref[...],
                               preferred_element_type=jnp.float32) + bf2_ref[...]

    return pl.pallas_call(
        body,
        grid=(N_PAD // BLK,),
        in_specs=[
            pl.BlockSpec((NC, BLK, D), lambda i: (0, i, 0)),
            pl.BlockSpec((BLK, D), lambda i: (i, 0)),
            pl.BlockSpec((BLK, 1), lambda i: (i, 0)),
            pl.BlockSpec((1, D), lambda i: (0, 0)),
            pl.BlockSpec((D, 64), lambda i: (0, 0)),
            pl.BlockSpec((1, 64), lambda i: (0, 0)),
            pl.BlockSpec((64, 1), lambda i: (0, 0)),
            pl.BlockSpec((1, 1), lambda i: (0, 0)),
        ],
        out_specs=pl.BlockSpec((BLK, 1), lambda i: (i, 0)),
        out_shape=jax.ShapeDtypeStruct((N_PAD, 1), jnp.float32),
    )(acc, g2, dinv, b2, Wf1, bf1, Wf2, bf2)


def kernel(x, edge_index, W1, b1, W2, b2, Wf1, bf1, Wf2, bf2):
    ei = edge_index.astype(jnp.int32)
    pad_idx = N_REAL + (jnp.arange(E_PAD - E, dtype=jnp.int32) % (N_PAD - N_REAL))
    src = jnp.concatenate([ei[0], pad_idx])
    dst = jnp.concatenate([ei[1], pad_idx])
    x_pad = jnp.pad(x, ((0, N_PAD - N_REAL), (0, 0)))
    ones128 = jnp.ones((CHUNK, D), jnp.float32)
    zerosd = jnp.zeros((ZR, D), jnp.float32)

    deg = _deg_kernel(dst, ones128, zerosd).reshape(NC, N_PAD, D)
    g1, dinv = _stage_a(x_pad, deg, W1)
    acc1 = _prop_kernel(g1, src, dst, zerosd).reshape(NC, N_PAD, D)
    g2 = _stage_c(acc1, g1, dinv, b1.reshape(1, D), W2)
    acc2 = _prop_kernel(g2, src, dst, zerosd).reshape(NC, N_PAD, D)
    out = _head(acc2, g2, dinv, b2.reshape(1, D),
                Wf1, bf1.reshape(1, 64), Wf2, bf2.reshape(1, 1))
    return out[:N_REAL]


# speedup vs baseline: 20.8595x; 1.3430x over previous
import functools

import jax
import jax.numpy as jnp
from jax import lax
from jax.experimental import pallas as pl
from jax.experimental.pallas import tpu as pltpu
from jax.experimental.pallas import tpu_sc as plsc

N_REAL = 10000
N_PAD = 10240
D = 128
E = 320000
NC = 2
NS = 16
NW = NC * NS
CHUNK = 128
CPT = 80
E_PAD = NW * CPT * CHUNK
RPT = N_PAD // NS
ZR = 64
BLK = 1024

_MESH = plsc.VectorSubcoreMesh(core_axis_name="c", subcore_axis_name="s")


G2 = 8


@functools.partial(
    pl.kernel,
    mesh=_MESH,
    out_type=jax.ShapeDtypeStruct((NC * N_PAD, D), jnp.float32),
    scratch_types=[
        pltpu.VMEM((G2, CHUNK), jnp.int32),
        pltpu.VMEM((CHUNK, D), jnp.float32),
        pltpu.VMEM((ZR, D), jnp.float32),
        pltpu.VMEM_SHARED((N_PAD, D), jnp.float32),
        pltpu.SemaphoreType.DMA,
        pltpu.SemaphoreType.DMA,
    ],
)
def _deg_kernel(dst_hbm, ones_hbm, zeros_hbm, out_hbm,
                dstg, ones_v, zbuf, acc_sh, ss0, ss1):
    c = lax.axis_index("c")
    s = lax.axis_index("s")
    wid = s * NC + c
    base = s * RPT
    half = G2 // 2

    pltpu.sync_copy(ones_hbm, ones_v)
    pltpu.sync_copy(zeros_hbm, zbuf)
    for k in range(RPT // ZR):
        pltpu.sync_copy(zbuf, acc_sh.at[pl.ds(base + k * ZR, ZR)])
    plsc.subcore_barrier()

    def pair(j, _):
        @pl.when(j > 0)
        def _():
            pltpu.make_async_copy(ones_v, acc_sh.at[dstg.at[0]], ss0).wait()
            pltpu.make_async_copy(ones_v, acc_sh.at[dstg.at[1]], ss1).wait()

        @pl.when(j % half == 0)
        def _():
            row0 = wid * CPT + (j // half) * G2
            pltpu.sync_copy(dst_hbm.at[pl.ds(row0, G2)], dstg)

        r0 = (2 * j) % G2
        r1 = r0 + 1
        pltpu.async_copy(ones_v, acc_sh.at[dstg.at[r0]], ss0, add=True)
        pltpu.async_copy(ones_v, acc_sh.at[dstg.at[r1]], ss1, add=True)
        return 0

    lax.fori_loop(0, CPT // 2, pair, 0)
    pltpu.make_async_copy(ones_v, acc_sh.at[dstg.at[0]], ss0).wait()
    pltpu.make_async_copy(ones_v, acc_sh.at[dstg.at[1]], ss1).wait()
    plsc.subcore_barrier()
    for k in range(RPT // ZR):
        pltpu.sync_copy(acc_sh.at[pl.ds(base + k * ZR, ZR)], zbuf)
        pltpu.sync_copy(zbuf, out_hbm.at[pl.ds(c * N_PAD + base + k * ZR, ZR)])


@functools.partial(
    pl.kernel,
    mesh=_MESH,
    out_type=jax.ShapeDtypeStruct((NC * N_PAD, D), jnp.float32),
    scratch_types=[
        pltpu.VMEM((G2, CHUNK), jnp.int32),
        pltpu.VMEM((G2, CHUNK), jnp.int32),
        pltpu.VMEM((CHUNK, D), jnp.float32),
        pltpu.VMEM((CHUNK, D), jnp.float32),
        pltpu.VMEM((ZR, D), jnp.float32),
        pltpu.VMEM_SHARED((N_PAD, D), jnp.float32),
        pltpu.SemaphoreType.DMA,
        pltpu.SemaphoreType.DMA,
        pltpu.SemaphoreType.DMA,
        pltpu.SemaphoreType.DMA,
    ],
)
def _prop_kernel(g_hbm, src_hbm, dst_hbm, zeros_hbm, out_hbm,
                 srcg, dstg, buf0, buf1, zbuf,
                 acc_sh, gs0, gs1, ss0, ss1):
    c = lax.axis_index("c")
    s = lax.axis_index("s")
    wid = s * NC + c
    base = s * RPT
    half = G2 // 2

    pltpu.sync_copy(zeros_hbm, zbuf)
    for k in range(RPT // ZR):
        pltpu.sync_copy(zbuf, acc_sh.at[pl.ds(base + k * ZR, ZR)])
    plsc.subcore_barrier()

    def pair(j, _):
        @pl.when(j > 0)
        def _():
            pltpu.make_async_copy(buf0, acc_sh.at[dstg.at[0]], ss0).wait()
            pltpu.make_async_copy(buf1, acc_sh.at[dstg.at[1]], ss1).wait()

        @pl.when(j % half == 0)
        def _():
            row0 = wid * CPT + (j // half) * G2
            pltpu.sync_copy(src_hbm.at[pl.ds(row0, G2)], srcg)
            pltpu.sync_copy(dst_hbm.at[pl.ds(row0, G2)], dstg)

        r0 = (2 * j) % G2
        r1 = r0 + 1
        h0 = pltpu.async_copy(g_hbm.at[srcg.at[r0]], buf0, gs0)
        h1 = pltpu.async_copy(g_hbm.at[srcg.at[r1]], buf1, gs1)
        h0.wait()
        pltpu.async_copy(buf0, acc_sh.at[dstg.at[r0]], ss0, add=True)
        h1.wait()
        pltpu.async_copy(buf1, acc_sh.at[dstg.at[r1]], ss1, add=True)
        return 0

    lax.fori_loop(0, CPT // 2, pair, 0)
    pltpu.make_async_copy(buf0, acc_sh.at[dstg.at[0]], ss0).wait()
    pltpu.make_async_copy(buf1, acc_sh.at[dstg.at[1]], ss1).wait()
    plsc.subcore_barrier()
    for k in range(RPT // ZR):
        pltpu.sync_copy(acc_sh.at[pl.ds(base + k * ZR, ZR)], zbuf)
        pltpu.sync_copy(zbuf, out_hbm.at[pl.ds(c * N_PAD + base + k * ZR, ZR)])


def _stage_a(x_pad, deg, W1):
    def body(x_ref, deg_ref, w_ref, g_ref, dinv_ref):
        i = pl.program_id(0)
        deg_tot = deg_ref[0, :, 0:1] + deg_ref[1, :, 0:1] + 1.0
        rows = i * BLK + lax.broadcasted_iota(jnp.int32, (BLK, 1), 0)
        dinv = jnp.where(rows < N_REAL, lax.rsqrt(deg_tot), 0.0)
        g_ref[...] = jnp.dot(x_ref[...], w_ref[...],
                             preferred_element_type=jnp.float32) * dinv
        dinv_ref[...] = dinv

    return pl.pallas_call(
        body,
        grid=(N_PAD // BLK,),
        in_specs=[
            pl.BlockSpec((BLK, D), lambda i: (i, 0)),
            pl.BlockSpec((NC, BLK, D), lambda i: (0, i, 0)),
            pl.BlockSpec((D, D), lambda i: (0, 0)),
        ],
        out_specs=[
            pl.BlockSpec((BLK, D), lambda i: (i, 0)),
            pl.BlockSpec((BLK, 1), lambda i: (i, 0)),
        ],
        out_shape=[
            jax.ShapeDtypeStruct((N_PAD, D), jnp.float32),
            jax.ShapeDtypeStruct((N_PAD, 1), jnp.float32),
        ],
    )(x_pad, deg, W1)


def _stage_c(acc, g1, dinv, b1, W2):
    def body(acc_ref, g_ref, dinv_ref, b_ref, w_ref, out_ref):
        dv = dinv_ref[...]
        t = jnp.tanh((acc_ref[0] + acc_ref[1] + g_ref[...]) * dv + b_ref[...])
        out_ref[...] = jnp.dot(t, w_ref[...],
                               preferred_element_type=jnp.float32) * dv

    return pl.pallas_call(
        body,
        grid=(N_PAD // BLK,),
        in_specs=[
            pl.BlockSpec((NC, BLK, D), lambda i: (0, i, 0)),
            pl.BlockSpec((BLK, D), lambda i: (i, 0)),
            pl.BlockSpec((BLK, 1), lambda i: (i, 0)),
            pl.BlockSpec((1, D), lambda i: (0, 0)),
            pl.BlockSpec((D, D), lambda i: (0, 0)),
        ],
        out_specs=pl.BlockSpec((BLK, D), lambda i: (i, 0)),
        out_shape=jax.ShapeDtypeStruct((N_PAD, D), jnp.float32),
    )(acc, g1, dinv, b1, W2)


def _head(acc, g2, dinv, b2, Wf1, bf1, Wf2, bf2):
    def body(acc_ref, g_ref, dinv_ref, b2_ref, wf1_ref, bf1_ref,
             wf2_ref, bf2_ref, out_ref):
        dv = dinv_ref[...]
        t = jnp.tanh((acc_ref[0] + acc_ref[1] + g_ref[...]) * dv + b2_ref[...])
        f = jnp.tanh(jnp.dot(t, wf1_ref[...],
                             preferred_element_type=jnp.float32) + bf1_ref[...])
        out_ref[...] = jnp.dot(f, wf2_ref[...],
                               preferred_element_type=jnp.float32) + bf2_ref[...]

    return pl.pallas_call(
        body,
        grid=(N_PAD // BLK,),
        in_specs=[
            pl.BlockSpec((NC, BLK, D), lambda i: (0, i, 0)),
            pl.BlockSpec((BLK, D), lambda i: (i, 0)),
            pl.BlockSpec((BLK, 1), lambda i: (i, 0)),
            pl.BlockSpec((1, D), lambda i: (0, 0)),
            pl.BlockSpec((D, 64), lambda i: (0, 0)),
            pl.BlockSpec((1, 64), lambda i: (0, 0)),
            pl.BlockSpec((64, 1), lambda i: (0, 0)),
            pl.BlockSpec((1, 1), lambda i: (0, 0)),
        ],
        out_specs=pl.BlockSpec((BLK, 1), lambda i: (i, 0)),
        out_shape=jax.ShapeDtypeStruct((N_PAD, 1), jnp.float32),
    )(acc, g2, dinv, b2, Wf1, bf1, Wf2, bf2)


def kernel(x, edge_index, W1, b1, W2, b2, Wf1, bf1, Wf2, bf2):
    ei = edge_index.astype(jnp.int32)
    pad_idx = N_REAL + (jnp.arange(E_PAD - E, dtype=jnp.int32) % (N_PAD - N_REAL))
    src = jnp.concatenate([ei[0], pad_idx])
    dst = jnp.concatenate([ei[1], pad_idx])
    x_pad = jnp.pad(x, ((0, N_PAD - N_REAL), (0, 0)))
    ones128 = jnp.ones((CHUNK, D), jnp.float32)
    zerosd = jnp.zeros((ZR, D), jnp.float32)

    src2 = src.reshape(E_PAD // CHUNK, CHUNK)
    dst2 = dst.reshape(E_PAD // CHUNK, CHUNK)

    deg = _deg_kernel(dst2, ones128, zerosd).reshape(NC, N_PAD, D)
    g1, dinv = _stage_a(x_pad, deg, W1)
    acc1 = _prop_kernel(g1, src2, dst2, zerosd).reshape(NC, N_PAD, D)
    g2 = _stage_c(acc1, g1, dinv, b1.reshape(1, D), W2)
    acc2 = _prop_kernel(g2, src2, dst2, zerosd).reshape(NC, N_PAD, D)
    out = _head(acc2, g2, dinv, b2.reshape(1, D),
                Wf1, bf1.reshape(1, 64), Wf2, bf2.reshape(1, 1))
    return out[:N_REAL]
